# baseline (device time: 11081 ns/iter reference)
import jax
import jax.numpy as jnp
from jax import lax
from jax.experimental import pallas as pl
from jax.experimental.pallas import tpu as pltpu

N_DEV = 4


def kernel(A, B):
    m, k_local = A.shape
    _, n = B.shape
    m_blk = m // N_DEV

    def body(a_ref, b_ref, out_ref, a_bf, b_bf, send_ref, recv_ref,
             send_sems, recv_sems):
        my = lax.axis_index("i")

        a_bf[:, :] = a_ref[:, :].astype(jnp.bfloat16)
        b_bf[:, :] = b_ref[:, :].astype(jnp.bfloat16)

        barrier_sem = pltpu.get_barrier_semaphore()
        for off in range(1, N_DEV):
            peer = (my + off) % N_DEV
            pl.semaphore_signal(
                barrier_sem, inc=1,
                device_id=(peer,), device_id_type=pl.DeviceIdType.MESH,
            )
        pl.semaphore_wait(barrier_sem, N_DEV - 1)

        sends = []
        for off in range(1, N_DEV):
            d = (my + off) % N_DEV
            blk = jnp.dot(
                a_bf[pl.ds(d * m_blk, m_blk), :], b_bf[:, :],
                preferred_element_type=jnp.float32,
            )
            send_ref[off - 1] = blk.astype(jnp.bfloat16)
            rdma = pltpu.make_async_remote_copy(
                src_ref=send_ref.at[off - 1],
                dst_ref=recv_ref.at[my],
                send_sem=send_sems.at[off - 1],
                recv_sem=recv_sems.at[my],
                device_id=(d,),
                device_id_type=pl.DeviceIdType.MESH,
            )
            rdma.start()
            sends.append(rdma)

        out_ref[:, :] = jnp.dot(
            a_bf[pl.ds(my * m_blk, m_blk), :], b_bf[:, :],
            preferred_element_type=jnp.float32,
        )

        for off in range(N_DEV - 1, 0, -1):
            s = (my + off) % N_DEV
            recv = pltpu.make_async_remote_copy(
                src_ref=send_ref.at[0],
                dst_ref=recv_ref.at[s],
                send_sem=send_sems.at[off - 1],
                recv_sem=recv_sems.at[s],
                device_id=(s,),
                device_id_type=pl.DeviceIdType.MESH,
            )
            recv.wait_recv()
            out_ref[:, :] = out_ref[:, :] + recv_ref[s].astype(jnp.float32)

        for rdma in sends:
            rdma.wait_send()

    return pl.pallas_call(
        body,
        out_shape=jax.ShapeDtypeStruct((m_blk, n), jnp.float32),
        in_specs=[
            pl.BlockSpec(memory_space=pltpu.VMEM),
            pl.BlockSpec(memory_space=pltpu.VMEM),
        ],
        out_specs=pl.BlockSpec(memory_space=pltpu.VMEM),
        scratch_shapes=[
            pltpu.VMEM((m, k_local), jnp.bfloat16),
            pltpu.VMEM((k_local, n), jnp.bfloat16),
            pltpu.VMEM((N_DEV - 1, m_blk, n), jnp.bfloat16),
            pltpu.VMEM((N_DEV, m_blk, n), jnp.bfloat16),
            pltpu.SemaphoreType.DMA((N_DEV - 1,)),
            pltpu.SemaphoreType.DMA((N_DEV,)),
        ],
        compiler_params=pltpu.CompilerParams(collective_id=0),
    )(A, B)


# device time: 11001 ns/iter; 1.0073x vs baseline; 1.0073x over previous
import jax
import jax.numpy as jnp
from jax import lax
from jax.experimental import pallas as pl
from jax.experimental.pallas import tpu as pltpu

N_DEV = 4


def kernel(A, B):
    m, k_local = A.shape
    _, n = B.shape
    m_blk = m // N_DEV

    def body(a_ref, b_ref, out_ref, part_ref, send_ref, recv_ref,
             send_sems, recv_sems):
        my = lax.axis_index("i")

        part_ref[:, :] = jnp.dot(
            a_ref[:, :], b_ref[:, :], preferred_element_type=jnp.float32
        )

        barrier_sem = pltpu.get_barrier_semaphore()
        for off in range(1, N_DEV):
            peer = (my + off) % N_DEV
            pl.semaphore_signal(
                barrier_sem, inc=1,
                device_id=(peer,), device_id_type=pl.DeviceIdType.MESH,
            )
        pl.semaphore_wait(barrier_sem, N_DEV - 1)

        sends = []
        for off in range(1, N_DEV):
            d = (my + off) % N_DEV
            send_ref[off - 1] = part_ref[pl.ds(d * m_blk, m_blk), :].astype(
                jnp.bfloat16
            )
            rdma = pltpu.make_async_remote_copy(
                src_ref=send_ref.at[off - 1],
                dst_ref=recv_ref.at[my],
                send_sem=send_sems.at[off - 1],
                recv_sem=recv_sems.at[my],
                device_id=(d,),
                device_id_type=pl.DeviceIdType.MESH,
            )
            rdma.start()
            sends.append(rdma)

        out_ref[:, :] = part_ref[pl.ds(my * m_blk, m_blk), :]

        for off in range(N_DEV - 1, 0, -1):
            s = (my + off) % N_DEV
            recv = pltpu.make_async_remote_copy(
                src_ref=send_ref.at[0],
                dst_ref=recv_ref.at[s],
                send_sem=send_sems.at[off - 1],
                recv_sem=recv_sems.at[s],
                device_id=(s,),
                device_id_type=pl.DeviceIdType.MESH,
            )
            recv.wait_recv()
            out_ref[:, :] = out_ref[:, :] + recv_ref[s].astype(jnp.float32)

        for rdma in sends:
            rdma.wait_send()

    return pl.pallas_call(
        body,
        out_shape=jax.ShapeDtypeStruct((m_blk, n), jnp.float32),
        in_specs=[
            pl.BlockSpec(memory_space=pltpu.VMEM),
            pl.BlockSpec(memory_space=pltpu.VMEM),
        ],
        out_specs=pl.BlockSpec(memory_space=pltpu.VMEM),
        scratch_shapes=[
            pltpu.VMEM((m, n), jnp.float32),
            pltpu.VMEM((N_DEV - 1, m_blk, n), jnp.bfloat16),
            pltpu.VMEM((N_DEV, m_blk, n), jnp.bfloat16),
            pltpu.SemaphoreType.DMA((N_DEV - 1,)),
            pltpu.SemaphoreType.DMA((N_DEV,)),
        ],
        compiler_params=pltpu.CompilerParams(collective_id=0),
    )(A, B)


# device time: 10665 ns/iter; 1.0390x vs baseline; 1.0315x over previous
import jax
import jax.numpy as jnp
from jax import lax
from jax.experimental import pallas as pl
from jax.experimental.pallas import tpu as pltpu

N_DEV = 4


def kernel(A, B):
    m, k_local = A.shape
    _, n = B.shape
    m_blk = m // N_DEV

    def body(a_ref, b_ref, out_ref, part_ref, send_ref, recv_ref,
             send_sems, recv_sems):
        my = lax.axis_index("i")

        barrier_sem = pltpu.get_barrier_semaphore()
        for off in range(1, N_DEV):
            peer = (my + off) % N_DEV
            pl.semaphore_signal(
                barrier_sem, inc=1,
                device_id=(peer,), device_id_type=pl.DeviceIdType.MESH,
            )

        part_ref[:, :] = jnp.dot(
            a_ref[:, :], b_ref[:, :], preferred_element_type=jnp.float32
        )

        pl.semaphore_wait(barrier_sem, N_DEV - 1)

        sends = []
        for off in range(1, N_DEV):
            d = (my + off) % N_DEV
            send_ref[off - 1] = part_ref[pl.ds(d * m_blk, m_blk), :].astype(
                jnp.bfloat16
            )
            rdma = pltpu.make_async_remote_copy(
                src_ref=send_ref.at[off - 1],
                dst_ref=recv_ref.at[my],
                send_sem=send_sems.at[off - 1],
                recv_sem=recv_sems.at[my],
                device_id=(d,),
                device_id_type=pl.DeviceIdType.MESH,
            )
            rdma.start()
            sends.append(rdma)

        out_ref[:, :] = part_ref[pl.ds(my * m_blk, m_blk), :]

        for off in range(N_DEV - 1, 0, -1):
            s = (my + off) % N_DEV
            recv = pltpu.make_async_remote_copy(
                src_ref=send_ref.at[0],
                dst_ref=recv_ref.at[s],
                send_sem=send_sems.at[off - 1],
                recv_sem=recv_sems.at[s],
                device_id=(s,),
                device_id_type=pl.DeviceIdType.MESH,
            )
            recv.wait_recv()
            out_ref[:, :] = out_ref[:, :] + recv_ref[s].astype(jnp.float32)

        for rdma in sends:
            rdma.wait_send()

    return pl.pallas_call(
        body,
        out_shape=jax.ShapeDtypeStruct((m_blk, n), jnp.float32),
        in_specs=[
            pl.BlockSpec(memory_space=pltpu.VMEM),
            pl.BlockSpec(memory_space=pltpu.VMEM),
        ],
        out_specs=pl.BlockSpec(memory_space=pltpu.VMEM),
        scratch_shapes=[
            pltpu.VMEM((m, n), jnp.float32),
            pltpu.VMEM((N_DEV - 1, m_blk, n), jnp.bfloat16),
            pltpu.VMEM((N_DEV, m_blk, n), jnp.bfloat16),
            pltpu.SemaphoreType.DMA((N_DEV - 1,)),
            pltpu.SemaphoreType.DMA((N_DEV,)),
        ],
        compiler_params=pltpu.CompilerParams(collective_id=0),
    )(A, B)


# device time: 10608 ns/iter; 1.0446x vs baseline; 1.0054x over previous
import jax
import jax.numpy as jnp
from jax import lax
from jax.experimental import pallas as pl
from jax.experimental.pallas import tpu as pltpu

N_DEV = 4


def kernel(A, B):
    m, k_local = A.shape
    _, n = B.shape
    m_blk = m // N_DEV

    def body(a_ref, b_ref, out_ref, part_ref, part_bf, recv_ref,
             send_sems, recv_sems):
        my = lax.axis_index("i")

        barrier_sem = pltpu.get_barrier_semaphore()
        for off in range(1, N_DEV):
            peer = (my + off) % N_DEV
            pl.semaphore_signal(
                barrier_sem, inc=1,
                device_id=(peer,), device_id_type=pl.DeviceIdType.MESH,
            )

        part_ref[:, :] = jnp.dot(
            a_ref[:, :], b_ref[:, :], preferred_element_type=jnp.float32
        )
        part_bf[:, :] = part_ref[:, :].astype(jnp.bfloat16)

        pl.semaphore_wait(barrier_sem, N_DEV - 1)

        sends = []
        for off in range(1, N_DEV):
            d = (my + off) % N_DEV
            rdma = pltpu.make_async_remote_copy(
                src_ref=part_bf.at[pl.ds(d * m_blk, m_blk), :],
                dst_ref=recv_ref.at[my],
                send_sem=send_sems.at[off - 1],
                recv_sem=recv_sems.at[my],
                device_id=(d,),
                device_id_type=pl.DeviceIdType.MESH,
            )
            rdma.start()
            sends.append(rdma)

        out_ref[:, :] = part_ref[pl.ds(my * m_blk, m_blk), :]

        for off in range(N_DEV - 1, 0, -1):
            s = (my + off) % N_DEV
            recv = pltpu.make_async_remote_copy(
                src_ref=part_bf.at[pl.ds(0, m_blk), :],
                dst_ref=recv_ref.at[s],
                send_sem=send_sems.at[off - 1],
                recv_sem=recv_sems.at[s],
                device_id=(s,),
                device_id_type=pl.DeviceIdType.MESH,
            )
            recv.wait_recv()
            out_ref[:, :] = out_ref[:, :] + recv_ref[s].astype(jnp.float32)

        for rdma in sends:
            rdma.wait_send()

    return pl.pallas_call(
        body,
        out_shape=jax.ShapeDtypeStruct((m_blk, n), jnp.float32),
        in_specs=[
            pl.BlockSpec(memory_space=pltpu.VMEM),
            pl.BlockSpec(memory_space=pltpu.VMEM),
        ],
        out_specs=pl.BlockSpec(memory_space=pltpu.VMEM),
        scratch_shapes=[
            pltpu.VMEM((m, n), jnp.float32),
            pltpu.VMEM((m, n), jnp.bfloat16),
            pltpu.VMEM((N_DEV, m_blk, n), jnp.bfloat16),
            pltpu.SemaphoreType.DMA((N_DEV - 1,)),
            pltpu.SemaphoreType.DMA((N_DEV,)),
        ],
        compiler_params=pltpu.CompilerParams(collective_id=0),
    )(A, B)
